# baseline (speedup 1.0000x reference)
import functools

import jax
import jax.numpy as jnp
from jax.experimental import pallas as pl
from jax.experimental.pallas import tpu as pltpu

D_MODEL = 1024
D_FF = 2816
E = 8
T = 2048

FC = 1408
NC = D_FF // FC
TM = 256
NM = T // TM


def _router_kernel(xf_ref, wr_ref, gate_ref, aux_ref):
    xf = xf_ref[...]
    wr = wr_ref[...]
    logits = jax.lax.dot_general(
        xf, wr, (((1,), (1,)), ((), ())),
        preferred_element_type=jnp.float32,
        precision=jax.lax.Precision.DEFAULT)
    m = jnp.max(logits, axis=1, keepdims=True)
    ex = jnp.exp(logits - m)
    probs = ex / jnp.sum(ex, axis=1, keepdims=True)

    top1 = jnp.argmax(probs, axis=1, keepdims=True)
    p1 = jnp.max(probs, axis=1, keepdims=True)
    lane = jax.lax.broadcasted_iota(jnp.int32, (T, E), 1)
    masked = jnp.where(lane == top1, -1.0, probs)
    top2 = jnp.argmax(masked, axis=1, keepdims=True)
    p2 = jnp.max(masked, axis=1, keepdims=True)
    denom = p1 + p2 + 1e-9
    w1 = p1 / denom
    w2 = p2 / denom
    one1 = (lane == top1).astype(jnp.float32)
    one2 = (lane == top2).astype(jnp.float32)
    gate_ref[...] = one1 * w1 + one2 * w2

    importance = jnp.sum(probs, axis=0, keepdims=True) / T
    load = jnp.sum(one1, axis=0, keepdims=True) / T
    aux_ref[...] = jnp.sum(importance * load, axis=1, keepdims=True) * E


def _dense_expert_kernel(gate_ref, x_ref, w1_ref, w3_ref, w2_ref, out_ref):
    e = pl.program_id(0)
    c = pl.program_id(1)
    m = pl.program_id(2)

    @pl.when((e == 0) & (c == 0))
    def _():
        out_ref[pl.ds(m * TM, TM), :] = jnp.zeros((TM, D_MODEL), jnp.float32)

    x = x_ref[pl.ds(m * TM, TM), :]
    w1 = w1_ref[0].astype(jnp.bfloat16)
    w3 = w3_ref[0].astype(jnp.bfloat16)
    w2 = w2_ref[0].astype(jnp.bfloat16)
    a = jax.lax.dot_general(x, w1, (((1,), (1,)), ((), ())),
                            preferred_element_type=jnp.float32)
    b = jax.lax.dot_general(x, w3, (((1,), (1,)), ((), ())),
                            preferred_element_type=jnp.float32)
    h = (a * jax.nn.sigmoid(a) * b).astype(jnp.bfloat16)
    y = jax.lax.dot_general(h, w2, (((1,), (1,)), ((), ())),
                            preferred_element_type=jnp.float32)
    g = gate_ref[0].reshape(TM, 1)
    out_ref[pl.ds(m * TM, TM), :] += g * y


@jax.jit
def _moe(x, W_router, W1, W3, W2):
    xf = x.reshape(T, D_MODEL)
    gate, aux = pl.pallas_call(
        _router_kernel,
        out_shape=(jax.ShapeDtypeStruct((T, E), jnp.float32),
                   jax.ShapeDtypeStruct((1, 1), jnp.float32)),
    )(xf, W_router)

    gate_b = gate.T.reshape(E, T, 1)
    xbf = xf.astype(jnp.bfloat16)

    out = pl.pallas_call(
        _dense_expert_kernel,
        grid=(E, NC, NM),
        in_specs=[
            pl.BlockSpec((1, TM, 1), lambda e, c, m: (e, m, 0)),
            pl.BlockSpec((T, D_MODEL), lambda e, c, m: (0, 0)),
            pl.BlockSpec((1, FC, D_MODEL), lambda e, c, m: (e, c, 0)),
            pl.BlockSpec((1, FC, D_MODEL), lambda e, c, m: (e, c, 0)),
            pl.BlockSpec((1, D_MODEL, FC), lambda e, c, m: (e, 0, c)),
        ],
        out_specs=pl.BlockSpec((T, D_MODEL), lambda e, c, m: (0, 0)),
        out_shape=jax.ShapeDtypeStruct((T, D_MODEL), jnp.float32),
        compiler_params=pltpu.CompilerParams(
            dimension_semantics=("arbitrary", "arbitrary", "arbitrary")),
    )(gate_b, xbf, W1, W3, W2)

    return (out.reshape(x.shape).astype(x.dtype),
            jnp.zeros((), jnp.float32),
            aux.reshape(()))


def kernel(x, W_router, W1, W3, W2):
    return _moe(x, W_router, W1, W3, W2)


# speedup vs baseline: 1.6969x; 1.6969x over previous
import functools

import numpy as np
import jax
import jax.numpy as jnp
from jax import lax
from jax.experimental import pallas as pl
from jax.experimental.pallas import tpu as pltpu
from jax.experimental.pallas import tpu_sc as plsc

D = 1024
F = 2816
E = 8
T = 2048
P = 2 * T

TM = 256
NT = P // TM + E
TOTPAD = NT * TM
FC = 1408
NC = F // FC

_HI = jax.lax.Precision.HIGHEST
_DF = jax.lax.Precision.DEFAULT

_j = np.arange(1024)
_ji, _je = _j // 8, _j % 8
M_PREFIX = ((_je[:, None] == _je[None, :]) & (_ji[:, None] < _ji[None, :])).astype(np.float32)
M_SUM = (_je[:, None] == np.arange(8)[None, :]).astype(np.float32)
REP_I = (np.arange(128)[:, None] == _ji[None, :]).astype(np.float32)
REP_E = (np.arange(8)[:, None] == _je[None, :]).astype(np.float32)
REP_IT = REP_I.T.copy()
T32 = (np.arange(32)[None, :] < np.arange(32)[:, None]).T.astype(np.float32)
T8 = (np.arange(8)[:, None] < np.arange(8)[None, :]).astype(np.float32)
I8 = np.eye(8, dtype=np.float32)


def _router_kernel(xf_ref, wr_ref, mpre_ref, msum_ref, repi_ref, repe_ref,
                   repit_ref, t32_ref, t8_ref, i8_ref,
                   g1_ref, g2_ref, dest_ref, te_ref, act_ref, aux_ref):
    xf = xf_ref[...]
    wr = wr_ref[...]
    logits = lax.dot_general(xf, wr, (((1,), (1,)), ((), ())),
                             preferred_element_type=jnp.float32,
                             precision=_DF)
    l3 = logits.reshape(16, 128, E)
    m = jnp.max(l3, axis=2, keepdims=True)
    ex = jnp.exp(l3 - m)
    probs = ex / jnp.sum(ex, axis=2, keepdims=True)

    top1 = jnp.argmax(probs, axis=2)
    p1 = jnp.max(probs, axis=2)
    lane3 = lax.broadcasted_iota(jnp.int32, (16, 128, E), 2)
    masked = jnp.where(lane3 == top1[..., None], -1.0, probs)
    top2 = jnp.argmax(masked, axis=2)
    p2 = jnp.max(masked, axis=2)
    denom = p1 + p2 + 1e-9
    g1_ref[...] = jnp.broadcast_to((p1 / denom)[..., None], (16, 128, 16))
    g2_ref[...] = jnp.broadcast_to((p2 / denom)[..., None], (16, 128, 16))

    importance = jnp.sum(probs, axis=(0, 1)).reshape(1, E) / T
    load = jnp.sum((lane3 == top1[..., None]).astype(jnp.float32),
                   axis=(0, 1)).reshape(1, E) / T
    aux_ref[...] = jnp.sum(importance * load).reshape(1, 1) * E

    e2 = jnp.concatenate([top1, top2], axis=0).astype(jnp.float32)
    e2rep = lax.dot_general(e2, repi_ref[...], (((1,), (0,)), ((), ())),
                            preferred_element_type=jnp.float32, precision=_DF)
    lane_e = lax.broadcasted_iota(jnp.int32, (32, 1024), 1) % 8
    o2 = (e2rep.astype(jnp.int32) == lane_e).astype(jnp.float32)
    s2 = lax.dot_general(o2, mpre_ref[...], (((1,), (0,)), ((), ())),
                         preferred_element_type=jnp.float32, precision=_DF)
    bsum = lax.dot_general(o2, msum_ref[...], (((1,), (0,)), ((), ())),
                           preferred_element_type=jnp.float32, precision=_DF)
    cblk = lax.dot_general(t32_ref[...], bsum, (((1,), (0,)), ((), ())),
                           preferred_element_type=jnp.float32, precision=_HI)
    counts = jnp.sum(bsum, axis=0, keepdims=True)
    padded = ((counts.astype(jnp.int32) + (TM - 1)) // TM * TM).astype(jnp.float32)
    base = lax.dot_general(padded, t8_ref[...], (((1,), (0,)), ((), ())),
                           preferred_element_type=jnp.float32, precision=_HI)
    crep = lax.dot_general(cblk, repe_ref[...], (((1,), (0,)), ((), ())),
                           preferred_element_type=jnp.float32, precision=_HI)
    brep = lax.dot_general(base, repe_ref[...], (((1,), (0,)), ((), ())),
                           preferred_element_type=jnp.float32, precision=_HI)
    dest2 = (s2 + crep + brep) * o2
    dsel = lax.dot_general(dest2, repit_ref[...], (((1,), (0,)), ((), ())),
                           preferred_element_type=jnp.float32, precision=_HI)
    dest_ref[...] = dsel.astype(jnp.int32)

    ends = base + padded
    ends_bc = lax.dot_general(jnp.ones((8, 1), jnp.float32), ends,
                              (((1,), (0,)), ((), ())),
                              preferred_element_type=jnp.float32, precision=_HI)
    ends_t = jnp.sum(ends_bc * i8_ref[...], axis=1, keepdims=True)
    iota_t = (lax.broadcasted_iota(jnp.int32, (E, NT), 1) * TM).astype(jnp.float32)
    te = jnp.sum((iota_t >= ends_t).astype(jnp.int32), axis=0, keepdims=True)
    te_ref[...] = jnp.minimum(te, E - 1)
    act_ref[...] = (iota_t[0:1, :] < ends[:, E - 1:E]).astype(jnp.int32)


def _expert_kernel(te_ref, act_ref, xs_ref, w1_ref, w3_ref, w2_ref,
                   out_ref, yacc_ref):
    c = pl.program_id(0)
    i = pl.program_id(1)

    @pl.when(act_ref[i] == 1)
    def _():
        x = xs_ref[...].astype(jnp.bfloat16)
        w1 = w1_ref[0].astype(jnp.bfloat16)
        w3 = w3_ref[0].astype(jnp.bfloat16)
        w2 = w2_ref[0].astype(jnp.bfloat16)
        a = lax.dot_general(x, w1, (((1,), (1,)), ((), ())),
                            preferred_element_type=jnp.float32)
        b = lax.dot_general(x, w3, (((1,), (1,)), ((), ())),
                            preferred_element_type=jnp.float32)
        h = (a * jax.nn.sigmoid(a) * b).astype(jnp.bfloat16)
        y = lax.dot_general(h, w2, (((1,), (1,)), ((), ())),
                            preferred_element_type=jnp.float32)

        @pl.when(c == 0)
        def _():
            yacc_ref[pl.ds(i * TM, TM), :] = y.astype(jnp.bfloat16)

        @pl.when(c == NC - 1)
        def _():
            out_ref[...] = yacc_ref[pl.ds(i * TM, TM), :].astype(jnp.float32) + y


@functools.lru_cache(maxsize=1)
def _sc_kernels():
    mesh = plsc.VectorSubcoreMesh(core_axis_name="c", subcore_axis_name="s")

    @functools.partial(
        pl.kernel, mesh=mesh,
        out_type=jax.ShapeDtypeStruct((TOTPAD, D), jnp.float32),
        scratch_types=[
            pltpu.VMEM((64,), jnp.int32),
            pltpu.VMEM((64,), jnp.int32),
            pltpu.VMEM((64, D), jnp.float32),
            pltpu.SemaphoreType.DMA,
            pltpu.SemaphoreType.DMA,
        ],
    )
    def _sc_dispatch(tok_hbm, dest_hbm, xf_hbm, xs_hbm, tok_v, dest_v, rows_v,
                     sem_g, sem_s):
        wid = lax.axis_index("s") * 2 + lax.axis_index("c")
        for j in range(2):
            base = wid * 128 + j * 64
            pltpu.sync_copy(tok_hbm.at[pl.ds(base, 64)], tok_v)
            pltpu.sync_copy(dest_hbm.at[pl.ds(base, 64)], dest_v)
            pltpu.async_copy(xf_hbm.at[tok_v], rows_v, sem_g).wait()
            pltpu.async_copy(rows_v, xs_hbm.at[dest_v], sem_s).wait()

    @functools.partial(
        pl.kernel, mesh=mesh,
        out_type=jax.ShapeDtypeStruct((T, D), jnp.float32),
        scratch_types=[
            pltpu.VMEM((32,), jnp.int32),
            pltpu.VMEM((32,), jnp.int32),
            pltpu.VMEM((32, 16), jnp.float32),
            pltpu.VMEM((32, 16), jnp.float32),
            pltpu.VMEM((32, D), jnp.float32),
            pltpu.VMEM((32, D), jnp.float32),
            pltpu.VMEM((32, D), jnp.float32),
            pltpu.SemaphoreType.DMA,
            pltpu.SemaphoreType.DMA,
        ],
    )
    def _sc_combine(ys_hbm, dest_hbm, g1_hbm, g2_hbm, out_hbm,
                    i1_v, i2_v, g1_v, g2_v, r1_v, r2_v, ob_v, sem1, sem2):
        wid = lax.axis_index("s") * 2 + lax.axis_index("c")
        for j in range(2):
            t0 = wid * 64 + j * 32
            pltpu.sync_copy(dest_hbm.at[pl.ds(t0, 32)], i1_v)
            pltpu.sync_copy(dest_hbm.at[pl.ds(T + t0, 32)], i2_v)
            pltpu.sync_copy(g1_hbm.at[pl.ds(t0, 32)], g1_v)
            pltpu.sync_copy(g2_hbm.at[pl.ds(t0, 32)], g2_v)
            pltpu.async_copy(ys_hbm.at[i1_v], r1_v, sem1).wait()
            pltpu.async_copy(ys_hbm.at[i2_v], r2_v, sem2).wait()
            for r in range(32):
                ga = g1_v[r]
                gb = g2_v[r]

                def body(kk, _):
                    sl = pl.ds(kk * 16, 16)
                    ob_v[r, sl] = ga * r1_v[r, sl] + gb * r2_v[r, sl]
                    return 0

                lax.fori_loop(0, D // 16, body, 0)
            pltpu.sync_copy(ob_v, out_hbm.at[pl.ds(t0, 32)])

    return _sc_dispatch, _sc_combine


@jax.jit
def _moe(x, W_router, W1, W3, W2):
    xf = x.reshape(T, D)
    g1, g2, dest, te, act, aux = pl.pallas_call(
        _router_kernel,
        out_shape=(
            jax.ShapeDtypeStruct((16, 128, 16), jnp.float32),
            jax.ShapeDtypeStruct((16, 128, 16), jnp.float32),
            jax.ShapeDtypeStruct((32, 128), jnp.int32),
            jax.ShapeDtypeStruct((1, NT), jnp.int32),
            jax.ShapeDtypeStruct((1, NT), jnp.int32),
            jax.ShapeDtypeStruct((1, 1), jnp.float32),
        ),
    )(xf, W_router, M_PREFIX, M_SUM, REP_I, REP_E, REP_IT, T32, T8, I8)

    sc_dispatch, sc_combine = _sc_kernels()
    tok = jnp.tile(jnp.arange(T, dtype=jnp.int32), 2)
    xs = sc_dispatch(tok, dest.reshape(P), xf)

    ys = pl.pallas_call(
        _expert_kernel,
        grid_spec=pltpu.PrefetchScalarGridSpec(
            num_scalar_prefetch=2,
            grid=(NC, NT),
            in_specs=[
                pl.BlockSpec((TM, D),
                             lambda c, i, te, act: (jnp.where(act[i] == 1, i, 0), 0)),
                pl.BlockSpec((1, FC, D), lambda c, i, te, act: (te[i], c, 0)),
                pl.BlockSpec((1, FC, D), lambda c, i, te, act: (te[i], c, 0)),
                pl.BlockSpec((1, D, FC), lambda c, i, te, act: (te[i], 0, c)),
            ],
            out_specs=pl.BlockSpec((TM, D), lambda c, i, te, act: (i, 0)),
            scratch_shapes=[pltpu.VMEM((TOTPAD, D), jnp.bfloat16)],
        ),
        out_shape=jax.ShapeDtypeStruct((TOTPAD, D), jnp.float32),
        compiler_params=pltpu.CompilerParams(
            dimension_semantics=("arbitrary", "arbitrary")),
    )(te.reshape(NT), act.reshape(NT), xs, W1, W3, W2)

    out = sc_combine(ys, dest.reshape(P), g1.reshape(T, 16), g2.reshape(T, 16))

    return (out.reshape(x.shape).astype(x.dtype),
            jnp.zeros((), jnp.float32),
            aux.reshape(()))


def kernel(x, W_router, W1, W3, W2):
    return _moe(x, W_router, W1, W3, W2)
